# baseline (speedup 1.0000x reference)
import jax
import jax.numpy as jnp
from jax import lax
from jax.experimental import pallas as pl
from jax.experimental.pallas import tpu as pltpu

_BLOCK = 1000


def _body(x_ref, y_ref, sm_ref, out_ref):
    x = x_ref[...]
    y = y_ref[...]
    sm = jnp.transpose(sm_ref[0])
    idx = sm[:, 4:5].astype(jnp.int32)

    B, P = x.shape

    dxp = x[:, 1:] - x[:, :-1]
    dyp = y[:, 1:] - y[:, :-1]
    sp = jnp.sqrt(dxp * dxp + dyp * dyp)
    z = x[:, :-1] * dxp + y[:, :-1] * dyp

    k = lax.broadcasted_iota(jnp.int32, (1, P - 1), 1)
    s_base = jnp.sum(jnp.where(k < idx, sp, 0.0), axis=1, keepdims=True)
    g = k == idx
    vx = jnp.sum(jnp.where(g, dxp, 0.0), axis=1, keepdims=True)
    vy = jnp.sum(jnp.where(g, dyp, 0.0), axis=1, keepdims=True)
    q = jnp.sum(jnp.where(g, z, 0.0), axis=1, keepdims=True)

    inv = 1.0 / jnp.sqrt(vx * vx + vy * vy)
    ux = vx * inv
    uy = vy * inv

    px = sm[:, 0:1]
    py = sm[:, 1:2]
    dx = sm[:, 2:3]
    dy = sm[:, 3:4]

    s = s_base + (px * vx + py * vy - q) * inv
    l = dx * uy - dy * ux
    out_ref[0] = jnp.transpose(jnp.concatenate([s, l], axis=1))


def kernel(proj_pt, dist, idx_before, idx_after, lane_features):
    del idx_after
    N, P, C = lane_features.shape
    x = lane_features[:, :, 0]
    y = lane_features[:, :, 1]
    nb = N // _BLOCK
    sm = jnp.concatenate(
        [
            jnp.transpose(proj_pt),
            jnp.transpose(dist),
            idx_before.astype(jnp.float32).reshape(1, N),
        ],
        axis=0,
    )
    sm3 = jnp.swapaxes(sm.reshape(5, nb, _BLOCK), 0, 1)

    out = pl.pallas_call(
        _body,
        grid=(nb,),
        in_specs=[
            pl.BlockSpec((_BLOCK, P), lambda i: (i, 0)),
            pl.BlockSpec((_BLOCK, P), lambda i: (i, 0)),
            pl.BlockSpec((1, 5, _BLOCK), lambda i: (i, 0, 0)),
        ],
        out_specs=pl.BlockSpec((1, 2, _BLOCK), lambda i: (i, 0, 0)),
        out_shape=jax.ShapeDtypeStruct((nb, 2, _BLOCK), jnp.float32),
        compiler_params=pltpu.CompilerParams(
            dimension_semantics=("arbitrary",),
        ),
    )(x, y, sm3)
    return jnp.swapaxes(out, 1, 2).reshape(N, 2)


# speedup vs baseline: 1.0335x; 1.0335x over previous
import jax
import jax.numpy as jnp
from jax import lax
from jax.experimental import pallas as pl
from jax.experimental.pallas import tpu as pltpu

_BLOCK = 2000


def _body(x_ref, y_ref, sm_ref, out_ref):
    x = x_ref[...]
    y = y_ref[...]
    sm = jnp.transpose(sm_ref[0])
    idx = sm[:, 4:5].astype(jnp.int32)

    B, P = x.shape

    dxp = x[:, 1:] - x[:, :-1]
    dyp = y[:, 1:] - y[:, :-1]
    sp = jnp.sqrt(dxp * dxp + dyp * dyp)

    k = lax.broadcasted_iota(jnp.int32, (1, P - 1), 1)
    s_base = jnp.sum(jnp.where(k < idx, sp, 0.0), axis=1, keepdims=True)

    c = lax.broadcasted_iota(jnp.int32, (1, P), 1)
    mb = c == idx
    ma = c == idx + 1
    xb = jnp.sum(jnp.where(mb, x, 0.0), axis=1, keepdims=True)
    yb = jnp.sum(jnp.where(mb, y, 0.0), axis=1, keepdims=True)
    xa = jnp.sum(jnp.where(ma, x, 0.0), axis=1, keepdims=True)
    ya = jnp.sum(jnp.where(ma, y, 0.0), axis=1, keepdims=True)

    vx = xa - xb
    vy = ya - yb
    mag = jnp.sqrt(vx * vx + vy * vy)
    ux = vx / mag
    uy = vy / mag

    px = sm[:, 0:1]
    py = sm[:, 1:2]
    dx = sm[:, 2:3]
    dy = sm[:, 3:4]

    s = s_base + (px - xb) * ux + (py - yb) * uy
    l = dx * uy - dy * ux
    out_ref[0] = jnp.transpose(jnp.concatenate([s, l], axis=1))


def kernel(proj_pt, dist, idx_before, idx_after, lane_features):
    del idx_after
    N, P, C = lane_features.shape
    x = lane_features[:, :, 0]
    y = lane_features[:, :, 1]
    nb = N // _BLOCK
    sm = jnp.concatenate(
        [
            jnp.transpose(proj_pt),
            jnp.transpose(dist),
            idx_before.astype(jnp.float32).reshape(1, N),
        ],
        axis=0,
    )
    sm3 = jnp.swapaxes(sm.reshape(5, nb, _BLOCK), 0, 1)

    out = pl.pallas_call(
        _body,
        grid=(nb,),
        in_specs=[
            pl.BlockSpec((_BLOCK, P), lambda i: (i, 0)),
            pl.BlockSpec((_BLOCK, P), lambda i: (i, 0)),
            pl.BlockSpec((1, 5, _BLOCK), lambda i: (i, 0, 0)),
        ],
        out_specs=pl.BlockSpec((1, 2, _BLOCK), lambda i: (i, 0, 0)),
        out_shape=jax.ShapeDtypeStruct((nb, 2, _BLOCK), jnp.float32),
        compiler_params=pltpu.CompilerParams(
            dimension_semantics=("arbitrary",),
        ),
    )(x, y, sm3)
    return jnp.swapaxes(out, 1, 2).reshape(N, 2)
